# baseline (speedup 1.0000x reference)
import functools

import jax
import jax.numpy as jnp
from jax.experimental import pallas as pl
from jax.experimental.pallas import tpu as pltpu

S, D, H, E = 2048, 1024, 1024, 16
TM = 256
NT = S // TM


def _dense_body(mask16_ref, x32_ref, wg_ref, xb_ref, W1_ref, b1_ref, W2_ref,
                b2_ref, out_ref, gates_ref):
    e = pl.program_id(0)
    i = pl.program_id(1)
    tok = pl.ds(i * TM, TM)

    @pl.when(e == 0)
    def _gating():
        xt = x32_ref[tok, :]
        logits = jnp.dot(xt, wg_ref[...], preferred_element_type=jnp.float32)
        idx = jax.lax.broadcasted_iota(jnp.int32, logits.shape, 1)
        m0 = jnp.max(logits, axis=1, keepdims=True)
        i0 = jnp.min(jnp.where(logits == m0, idx, E), axis=1, keepdims=True)
        l2 = jnp.where(idx == i0, -jnp.inf, logits)
        m1 = jnp.max(l2, axis=1, keepdims=True)
        i1 = jnp.min(jnp.where(l2 == m1, idx, E), axis=1, keepdims=True)
        g0 = 1.0 / (1.0 + jnp.exp(m1 - m0))
        g1 = 1.0 - g0
        gates = jnp.where(idx == i0, g0, 0.0) + jnp.where(idx == i1, g1, 0.0)
        gates_ref[tok, :] = gates * mask16_ref[tok, :]
        out_ref[tok, :] = xt

    xb = xb_ref[tok, :]
    h = jnp.dot(xb, W1_ref[0], preferred_element_type=jnp.float32) + b1_ref[0]
    h = jnp.maximum(h, 0.0).astype(jnp.bfloat16)
    o = jnp.dot(h, W2_ref[0], preferred_element_type=jnp.float32) + b2_ref[0]
    idx = jax.lax.broadcasted_iota(jnp.int32, (TM, E), 1)
    g = jnp.sum(jnp.where(idx == e, gates_ref[tok, :], 0.0), axis=1,
                keepdims=True)
    out_ref[tok, :] += g * o


@jax.jit
def _moe(x, mask, w_gate, W1, b1, W2, b2):
    x2 = x.reshape(S, D)
    mask16 = jnp.broadcast_to(
        mask.astype(jnp.float32).reshape(S, 1), (S, E))
    xb = x2.astype(jnp.bfloat16)
    W1b = W1.astype(jnp.bfloat16)
    W2b = W2.astype(jnp.bfloat16)
    b1r = b1.reshape(E, 1, H)
    b2r = b2.reshape(E, 1, D)

    out = pl.pallas_call(
        _dense_body,
        grid=(E, NT),
        in_specs=[
            pl.BlockSpec((S, E), lambda e, i: (0, 0)),
            pl.BlockSpec((S, D), lambda e, i: (0, 0)),
            pl.BlockSpec((D, E), lambda e, i: (0, 0)),
            pl.BlockSpec((S, D), lambda e, i: (0, 0)),
            pl.BlockSpec((1, D, H), lambda e, i: (e, 0, 0)),
            pl.BlockSpec((1, 1, H), lambda e, i: (e, 0, 0)),
            pl.BlockSpec((1, H, D), lambda e, i: (e, 0, 0)),
            pl.BlockSpec((1, 1, D), lambda e, i: (e, 0, 0)),
        ],
        out_specs=pl.BlockSpec((S, D), lambda e, i: (0, 0)),
        out_shape=jax.ShapeDtypeStruct((S, D), jnp.float32),
        scratch_shapes=[pltpu.VMEM((S, E), jnp.float32)],
        compiler_params=pltpu.CompilerParams(
            dimension_semantics=("arbitrary", "arbitrary")),
    )(mask16, x2, w_gate, xb, W1b, b1r, W2b, b2r)
    return out.reshape(1, S, D), jnp.float32(0.0)


def kernel(x, mask, w_gate, W1, b1, W2, b2):
    return _moe(x, mask, w_gate, W1, b1, W2, b2)


# speedup vs baseline: 1.4104x; 1.4104x over previous
import functools

import jax
import jax.numpy as jnp
from jax import lax
from jax.experimental import pallas as pl
from jax.experimental.pallas import tpu as pltpu
from jax.experimental.pallas import tpu_sc as plsc

S, D, H, E = 2048, 1024, 1024, 16
TM = 256
NB = S * 2 // TM + E
PP = NB * TM
TT = 256
NTT = S // TT

NC, NS = 2, 16
NW = NC * NS
TPW = S // NW
SUB = 16

_mesh = plsc.VectorSubcoreMesh(core_axis_name="c", subcore_axis_name="s")


def _route_body(x_ref, wg_ref, mask_ref, pos0_ref, pos1_ref, be_ref, na_ref):
    lane = lambda shp: lax.broadcasted_iota(jnp.int32, shp, 1)
    tri = (lax.broadcasted_iota(jnp.int32, (TT, TT), 1)
           < lax.broadcasted_iota(jnp.int32, (TT, TT), 0)).astype(jnp.float32)

    def top2(ti):
        tok = pl.ds(ti * TT, TT)
        logits = jnp.dot(x_ref[tok, :], wg_ref[...],
                         preferred_element_type=jnp.float32)
        idx = lane(logits.shape)
        m0 = jnp.max(logits, axis=1, keepdims=True)
        i0 = jnp.min(jnp.where(logits == m0, idx, E), axis=1, keepdims=True)
        l2 = jnp.where(idx == i0, -1e30, logits)
        m1 = jnp.max(l2, axis=1, keepdims=True)
        i1 = jnp.min(jnp.where(l2 == m1, idx, E), axis=1, keepdims=True)
        return i0, i1

    carry = jnp.zeros((1, E), jnp.float32)
    rank0, oh0s, oh1s = [], [], []
    for ti in range(NTT):
        i0, i1 = top2(ti)
        oh0 = (lane((TT, E)) == i0).astype(jnp.float32)
        oh1 = (lane((TT, E)) == i1).astype(jnp.float32)
        cum = jnp.dot(tri, oh0, preferred_element_type=jnp.float32) + carry
        rank0.append(jnp.sum(jnp.where(lane((TT, E)) == i0, cum, 0.0),
                             axis=1, keepdims=True))
        carry = carry + jnp.sum(oh0, axis=0, keepdims=True)
        oh0s.append((i0, oh0))
        oh1s.append((i1, oh1))
    rank1 = []
    for ti in range(NTT):
        i1, oh1 = oh1s[ti]
        cum = jnp.dot(tri, oh1, preferred_element_type=jnp.float32) + carry
        rank1.append(jnp.sum(jnp.where(lane((TT, E)) == i1, cum, 0.0),
                             axis=1, keepdims=True))
        carry = carry + jnp.sum(oh1, axis=0, keepdims=True)
    cnt = carry
    nb = jnp.floor((cnt + (TM - 1)) * (1.0 / TM))
    tri_e = (lax.broadcasted_iota(jnp.int32, (E, E), 0)
             < lax.broadcasted_iota(jnp.int32, (E, E), 1)).astype(jnp.float32)
    bstart = jnp.dot(nb, tri_e, preferred_element_type=jnp.float32)
    pad_off = bstart * TM
    for ti in range(NTT):
        i0, _ = oh0s[ti]
        i1, _ = oh1s[ti]
        off0 = jnp.sum(jnp.where(lane((TT, E)) == i0, pad_off, 0.0),
                       axis=1, keepdims=True)
        off1 = jnp.sum(jnp.where(lane((TT, E)) == i1, pad_off, 0.0),
                       axis=1, keepdims=True)
        tok = pl.ds(ti * TT, TT)
        pos0_ref[tok, :] = (off0 + rank0[ti]).astype(jnp.int32)
        pos1_ref[tok, :] = (off1 + rank1[ti]).astype(jnp.int32)
    bidx = lax.broadcasted_iota(jnp.int32, (NB, E), 0).astype(jnp.float32)
    bst = jnp.broadcast_to(bstart, (NB, E))
    be_ref[...] = (jnp.sum((bst <= bidx).astype(jnp.float32), axis=1,
                           keepdims=True) - 1.0).astype(jnp.int32)
    na_ref[...] = jnp.sum(nb, axis=1, keepdims=True).astype(jnp.int32)


def _route(x2, w_gate, mask16):
    return pl.pallas_call(
        _route_body,
        grid=(),
        in_specs=[
            pl.BlockSpec((S, D), lambda: (0, 0)),
            pl.BlockSpec((D, E), lambda: (0, 0)),
            pl.BlockSpec((S, E), lambda: (0, 0)),
        ],
        out_specs=[
            pl.BlockSpec((S, 1), lambda: (0, 0)),
            pl.BlockSpec((S, 1), lambda: (0, 0)),
            pl.BlockSpec((NB, 1), lambda: (0, 0)),
            pl.BlockSpec((1, 1), lambda: (0, 0)),
        ],
        out_shape=[
            jax.ShapeDtypeStruct((S, 1), jnp.int32),
            jax.ShapeDtypeStruct((S, 1), jnp.int32),
            jax.ShapeDtypeStruct((NB, 1), jnp.int32),
            jax.ShapeDtypeStruct((1, 1), jnp.int32),
        ],
    )(x2, w_gate, mask16)


@functools.partial(
    pl.kernel,
    out_type=jax.ShapeDtypeStruct((PP, D), jnp.float32),
    mesh=_mesh,
    scratch_types=[
        pltpu.VMEM((TPW,), jnp.int32),
        pltpu.VMEM((TPW,), jnp.int32),
        pltpu.VMEM((TPW, D), jnp.float32),
        pltpu.SemaphoreType.DMA,
        pltpu.SemaphoreType.DMA,
    ],
)
def _dispatch(x_hbm, pos0_hbm, pos1_hbm, xs_hbm, idx0_v, idx1_v, xbuf_v,
              sem0, sem1):
    wid = lax.axis_index("s") * NC + lax.axis_index("c")
    base = wid * TPW
    pltpu.sync_copy(pos0_hbm.at[pl.ds(base, TPW)], idx0_v)
    pltpu.sync_copy(pos1_hbm.at[pl.ds(base, TPW)], idx1_v)
    pltpu.sync_copy(x_hbm.at[pl.ds(base, TPW)], xbuf_v)
    c0 = pltpu.async_copy(xbuf_v, xs_hbm.at[idx0_v], sem0)
    c1 = pltpu.async_copy(xbuf_v, xs_hbm.at[idx1_v], sem1)
    c0.wait()
    c1.wait()


def _gmm_body(be_ref, na_ref, xs_ref, wg_ref, W1_ref, b1_ref, W2_ref, b2_ref,
              out_ref):
    b = pl.program_id(0)

    @pl.when(b < na_ref[0])
    def _():
        xf = xs_ref[...]
        e = be_ref[b]
        logits = jnp.dot(xf, wg_ref[...], preferred_element_type=jnp.float32)
        idx = lax.broadcasted_iota(jnp.int32, logits.shape, 1)
        m0 = jnp.max(logits, axis=1, keepdims=True)
        i0 = jnp.min(jnp.where(logits == m0, idx, E), axis=1, keepdims=True)
        l2 = jnp.where(idx == i0, -1e30, logits)
        m1 = jnp.max(l2, axis=1, keepdims=True)
        i1 = jnp.min(jnp.where(l2 == m1, idx, E), axis=1, keepdims=True)
        g0 = 1.0 / (1.0 + jnp.exp(m1 - m0))
        g = jnp.where(i0 == e, g0, 0.0) + jnp.where(i1 == e, 1.0 - g0, 0.0)

        xb = xf.astype(jnp.bfloat16)
        h = jnp.dot(xb, W1_ref[0], preferred_element_type=jnp.float32)
        h = jnp.maximum(h + b1_ref[0], 0.0).astype(jnp.bfloat16)
        o = jnp.dot(h, W2_ref[0], preferred_element_type=jnp.float32)
        out_ref[...] = g * (o + b2_ref[0])


def _gmm(be, na, xs, w_gate, W1b, b1r, W2b, b2r):
    def _b(b, be, na):
        return jnp.minimum(b, na[0] - 1)

    grid_spec = pltpu.PrefetchScalarGridSpec(
        num_scalar_prefetch=2,
        grid=(NB,),
        in_specs=[
            pl.BlockSpec((TM, D), lambda b, be, na: (_b(b, be, na), 0)),
            pl.BlockSpec((D, E), lambda b, be, na: (0, 0)),
            pl.BlockSpec((1, D, H), lambda b, be, na: (be[_b(b, be, na)], 0, 0)),
            pl.BlockSpec((1, 1, H), lambda b, be, na: (be[_b(b, be, na)], 0, 0)),
            pl.BlockSpec((1, H, D), lambda b, be, na: (be[_b(b, be, na)], 0, 0)),
            pl.BlockSpec((1, 1, D), lambda b, be, na: (be[_b(b, be, na)], 0, 0)),
        ],
        out_specs=pl.BlockSpec(
            (TM, D), lambda b, be, na: (_b(b, be, na), 0)),
    )
    return pl.pallas_call(
        _gmm_body,
        grid_spec=grid_spec,
        out_shape=jax.ShapeDtypeStruct((PP, D), jnp.float32),
        compiler_params=pltpu.CompilerParams(
            dimension_semantics=("arbitrary",)),
    )(be, na, xs, w_gate, W1b, b1r, W2b, b2r)


@functools.partial(
    pl.kernel,
    out_type=jax.ShapeDtypeStruct((S, D), jnp.float32),
    mesh=_mesh,
    scratch_types=[
        pltpu.VMEM((SUB,), jnp.int32),
        pltpu.VMEM((SUB,), jnp.int32),
        pltpu.VMEM((SUB, D), jnp.float32),
        pltpu.VMEM((SUB, D), jnp.float32),
        pltpu.VMEM((SUB, D), jnp.float32),
        pltpu.VMEM((SUB, D), jnp.float32),
        pltpu.SemaphoreType.DMA,
        pltpu.SemaphoreType.DMA,
    ],
)
def _combine(os_hbm, x_hbm, pos0_hbm, pos1_hbm, y_hbm,
             idx0_v, idx1_v, xb_v, ab_v, bb_v, yb_v, sa, sb):
    wid = lax.axis_index("s") * NC + lax.axis_index("c")
    for j in range(TPW // SUB):
        base = wid * TPW + j * SUB
        pltpu.sync_copy(pos0_hbm.at[pl.ds(base, SUB)], idx0_v)
        pltpu.sync_copy(pos1_hbm.at[pl.ds(base, SUB)], idx1_v)
        ca = pltpu.async_copy(os_hbm.at[idx0_v], ab_v, sa)
        cb = pltpu.async_copy(os_hbm.at[idx1_v], bb_v, sb)
        pltpu.sync_copy(x_hbm.at[pl.ds(base, SUB)], xb_v)
        ca.wait()
        cb.wait()

        def col(c, _):
            sl = pl.ds(c * 16, 16)
            for t in range(SUB):
                yb_v[t, sl] = xb_v[t, sl] + ab_v[t, sl] + bb_v[t, sl]
            return _

        lax.fori_loop(0, D // 16, col, 0)
        pltpu.sync_copy(yb_v, y_hbm.at[pl.ds(base, SUB)])


@jax.jit
def _moe(x, mask, w_gate, W1, b1, W2, b2):
    x2 = x.reshape(S, D)
    mask16 = jnp.broadcast_to(mask.astype(jnp.float32).reshape(S, 1), (S, E))
    W1b = W1.astype(jnp.bfloat16)
    W2b = W2.astype(jnp.bfloat16)
    b1r = b1.reshape(E, 1, H)
    b2r = b2.reshape(E, 1, D)

    pos0, pos1, be, na = _route(x2, w_gate, mask16)
    pos0 = pos0.reshape(S)
    pos1 = pos1.reshape(S)
    be = be.reshape(NB)
    na = na.reshape(1)

    xs = _dispatch(x2, pos0, pos1)
    os_ = _gmm(be, na, xs, w_gate, W1b, b1r, W2b, b2r)
    y = _combine(os_, x2, pos0, pos1)
    return y.reshape(1, S, D), jnp.float32(0.0)


def kernel(x, mask, w_gate, W1, b1, W2, b2):
    return _moe(x, mask, w_gate, W1, b1, W2, b2)


# speedup vs baseline: 1.8208x; 1.2910x over previous
import functools

import jax
import jax.numpy as jnp
from jax import lax
from jax.experimental import pallas as pl
from jax.experimental.pallas import tpu as pltpu
from jax.experimental.pallas import tpu_sc as plsc

S, D, H, E = 2048, 1024, 1024, 16
TM = 256
NB = S * 2 // TM + E
PP = NB * TM
TT = 256
NTT = S // TT

NC, NS = 2, 16
NW = NC * NS
TPW = S // NW
SUB = 16

_mesh = plsc.VectorSubcoreMesh(core_axis_name="c", subcore_axis_name="s")


def _route_body(x_ref, wg_ref, pos0_ref, pos1_ref, be_ref, na_ref):
    lane = lambda shp: lax.broadcasted_iota(jnp.int32, shp, 1)
    tri = (lax.broadcasted_iota(jnp.int32, (TT, TT), 1)
           < lax.broadcasted_iota(jnp.int32, (TT, TT), 0)).astype(jnp.float32)

    def top2(ti):
        tok = pl.ds(ti * TT, TT)
        logits = jnp.dot(x_ref[tok, :], wg_ref[...],
                         preferred_element_type=jnp.float32)
        idx = lane(logits.shape)
        m0 = jnp.max(logits, axis=1, keepdims=True)
        i0 = jnp.min(jnp.where(logits == m0, idx, E), axis=1, keepdims=True)
        l2 = jnp.where(idx == i0, -1e30, logits)
        m1 = jnp.max(l2, axis=1, keepdims=True)
        i1 = jnp.min(jnp.where(l2 == m1, idx, E), axis=1, keepdims=True)
        return i0, i1

    carry = jnp.zeros((1, E), jnp.float32)
    rank0, oh0s, oh1s = [], [], []
    for ti in range(NTT):
        i0, i1 = top2(ti)
        oh0 = (lane((TT, E)) == i0).astype(jnp.float32)
        oh1 = (lane((TT, E)) == i1).astype(jnp.float32)
        cum = jnp.dot(tri, oh0, preferred_element_type=jnp.float32) + carry
        rank0.append(jnp.sum(jnp.where(lane((TT, E)) == i0, cum, 0.0),
                             axis=1, keepdims=True))
        carry = carry + jnp.sum(oh0, axis=0, keepdims=True)
        oh0s.append((i0, oh0))
        oh1s.append((i1, oh1))
    rank1 = []
    for ti in range(NTT):
        i1, oh1 = oh1s[ti]
        cum = jnp.dot(tri, oh1, preferred_element_type=jnp.float32) + carry
        rank1.append(jnp.sum(jnp.where(lane((TT, E)) == i1, cum, 0.0),
                             axis=1, keepdims=True))
        carry = carry + jnp.sum(oh1, axis=0, keepdims=True)
    cnt = carry
    nb = jnp.floor((cnt + (TM - 1)) * (1.0 / TM))
    tri_e = (lax.broadcasted_iota(jnp.int32, (E, E), 0)
             < lax.broadcasted_iota(jnp.int32, (E, E), 1)).astype(jnp.float32)
    bstart = jnp.dot(nb, tri_e, preferred_element_type=jnp.float32)
    pad_off = bstart * TM
    for ti in range(NTT):
        i0, _ = oh0s[ti]
        i1, _ = oh1s[ti]
        off0 = jnp.sum(jnp.where(lane((TT, E)) == i0, pad_off, 0.0),
                       axis=1, keepdims=True)
        off1 = jnp.sum(jnp.where(lane((TT, E)) == i1, pad_off, 0.0),
                       axis=1, keepdims=True)
        tok = pl.ds(ti * TT, TT)
        pos0_ref[tok, :] = (off0 + rank0[ti]).astype(jnp.int32)
        pos1_ref[tok, :] = (off1 + rank1[ti]).astype(jnp.int32)
    bidx = lax.broadcasted_iota(jnp.int32, (NB, E), 0).astype(jnp.float32)
    bst = jnp.broadcast_to(bstart, (NB, E))
    be_ref[...] = (jnp.sum((bst <= bidx).astype(jnp.float32), axis=1,
                           keepdims=True) - 1.0).astype(jnp.int32)
    na_ref[...] = jnp.sum(nb, axis=1, keepdims=True).astype(jnp.int32)


def _route(x2, w_gate):
    return pl.pallas_call(
        _route_body,
        grid=(),
        in_specs=[
            pl.BlockSpec((S, D), lambda: (0, 0)),
            pl.BlockSpec((D, E), lambda: (0, 0)),
        ],
        out_specs=[
            pl.BlockSpec((S, 1), lambda: (0, 0)),
            pl.BlockSpec((S, 1), lambda: (0, 0)),
            pl.BlockSpec((NB, 1), lambda: (0, 0)),
            pl.BlockSpec((1, 1), lambda: (0, 0)),
        ],
        out_shape=[
            jax.ShapeDtypeStruct((S, 1), jnp.int32),
            jax.ShapeDtypeStruct((S, 1), jnp.int32),
            jax.ShapeDtypeStruct((NB, 1), jnp.int32),
            jax.ShapeDtypeStruct((1, 1), jnp.int32),
        ],
    )(x2, w_gate)


@functools.partial(
    pl.kernel,
    out_type=jax.ShapeDtypeStruct((PP, D), jnp.float32),
    mesh=_mesh,
    scratch_types=[
        pltpu.VMEM((TPW,), jnp.int32),
        pltpu.VMEM((TPW,), jnp.int32),
        pltpu.VMEM((TPW, D), jnp.float32),
        pltpu.SemaphoreType.DMA,
        pltpu.SemaphoreType.DMA,
    ],
)
def _dispatch(x_hbm, pos0_hbm, pos1_hbm, xs_hbm, idx0_v, idx1_v, xbuf_v,
              sem0, sem1):
    wid = lax.axis_index("s") * NC + lax.axis_index("c")
    base = wid * TPW
    pltpu.sync_copy(pos0_hbm.at[pl.ds(base, TPW)], idx0_v)
    pltpu.sync_copy(pos1_hbm.at[pl.ds(base, TPW)], idx1_v)
    pltpu.sync_copy(x_hbm.at[pl.ds(base, TPW)], xbuf_v)
    c0 = pltpu.async_copy(xbuf_v, xs_hbm.at[idx0_v], sem0)
    c1 = pltpu.async_copy(xbuf_v, xs_hbm.at[idx1_v], sem1)
    c0.wait()
    c1.wait()


def _gmm_body(be_ref, na_ref, xs_ref, wg_ref, W1_ref, b1_ref, W2_ref, b2_ref,
              out_ref):
    b = pl.program_id(0)

    @pl.when(b < na_ref[0])
    def _():
        xf = xs_ref[...]
        e = be_ref[b]
        logits = jnp.dot(xf, wg_ref[...], preferred_element_type=jnp.float32)
        idx = lax.broadcasted_iota(jnp.int32, logits.shape, 1)
        m0 = jnp.max(logits, axis=1, keepdims=True)
        i0 = jnp.min(jnp.where(logits == m0, idx, E), axis=1, keepdims=True)
        l2 = jnp.where(idx == i0, -1e30, logits)
        m1 = jnp.max(l2, axis=1, keepdims=True)
        i1 = jnp.min(jnp.where(l2 == m1, idx, E), axis=1, keepdims=True)
        g0 = 1.0 / (1.0 + jnp.exp(m1 - m0))
        g = jnp.where(i0 == e, g0, 0.0) + jnp.where(i1 == e, 1.0 - g0, 0.0)

        xb = xf.astype(jnp.bfloat16)
        h = jnp.dot(xb, W1_ref[0].astype(jnp.bfloat16),
                    preferred_element_type=jnp.float32)
        h = jnp.maximum(h + b1_ref[0], 0.0).astype(jnp.bfloat16)
        o = jnp.dot(h, W2_ref[0].astype(jnp.bfloat16),
                    preferred_element_type=jnp.float32)
        out_ref[...] = g * (o + b2_ref[0])


def _gmm(be, na, xs, w_gate, W1b, b1r, W2b, b2r):
    def _b(b, be, na):
        return jnp.minimum(b, na[0] - 1)

    grid_spec = pltpu.PrefetchScalarGridSpec(
        num_scalar_prefetch=2,
        grid=(NB,),
        in_specs=[
            pl.BlockSpec((TM, D), lambda b, be, na: (_b(b, be, na), 0)),
            pl.BlockSpec((D, E), lambda b, be, na: (0, 0)),
            pl.BlockSpec((1, D, H), lambda b, be, na: (be[_b(b, be, na)], 0, 0)),
            pl.BlockSpec((1, 1, H), lambda b, be, na: (be[_b(b, be, na)], 0, 0)),
            pl.BlockSpec((1, H, D), lambda b, be, na: (be[_b(b, be, na)], 0, 0)),
            pl.BlockSpec((1, 1, D), lambda b, be, na: (be[_b(b, be, na)], 0, 0)),
        ],
        out_specs=pl.BlockSpec(
            (TM, D), lambda b, be, na: (_b(b, be, na), 0)),
    )
    return pl.pallas_call(
        _gmm_body,
        grid_spec=grid_spec,
        out_shape=jax.ShapeDtypeStruct((PP, D), jnp.float32),
        compiler_params=pltpu.CompilerParams(
            dimension_semantics=("arbitrary",)),
    )(be, na, xs, w_gate, W1b, b1r, W2b, b2r)


@functools.partial(
    pl.kernel,
    out_type=jax.ShapeDtypeStruct((S, D), jnp.float32),
    mesh=_mesh,
    scratch_types=[
        pltpu.VMEM((SUB,), jnp.int32),
        pltpu.VMEM((SUB,), jnp.int32),
        pltpu.VMEM((SUB, D), jnp.float32),
        pltpu.VMEM((SUB, D), jnp.float32),
        pltpu.VMEM((SUB, D), jnp.float32),
        pltpu.VMEM((SUB, D), jnp.float32),
        pltpu.SemaphoreType.DMA,
        pltpu.SemaphoreType.DMA,
    ],
)
def _combine(os_hbm, x_hbm, pos0_hbm, pos1_hbm, y_hbm,
             idx0_v, idx1_v, xb_v, ab_v, bb_v, yb_v, sa, sb):
    wid = lax.axis_index("s") * NC + lax.axis_index("c")
    for j in range(TPW // SUB):
        base = wid * TPW + j * SUB
        pltpu.sync_copy(pos0_hbm.at[pl.ds(base, SUB)], idx0_v)
        pltpu.sync_copy(pos1_hbm.at[pl.ds(base, SUB)], idx1_v)
        ca = pltpu.async_copy(os_hbm.at[idx0_v], ab_v, sa)
        cb = pltpu.async_copy(os_hbm.at[idx1_v], bb_v, sb)
        pltpu.sync_copy(x_hbm.at[pl.ds(base, SUB)], xb_v)
        ca.wait()
        cb.wait()

        def col(c, _):
            sl = pl.ds(c * 16, 16)
            for t in range(SUB):
                yb_v[t, sl] = xb_v[t, sl] + ab_v[t, sl] + bb_v[t, sl]
            return _

        lax.fori_loop(0, D // 16, col, 0)
        pltpu.sync_copy(yb_v, y_hbm.at[pl.ds(base, SUB)])


@jax.jit
def _moe(x, mask, w_gate, W1, b1, W2, b2):
    x2 = x.reshape(S, D)
    b1r = b1.reshape(E, 1, H)
    b2r = b2.reshape(E, 1, D)

    pos0, pos1, be, na = _route(x2, w_gate)
    pos0 = pos0.reshape(S)
    pos1 = pos1.reshape(S)
    be = be.reshape(NB)
    na = na.reshape(1)

    xs = _dispatch(x2, pos0, pos1)
    os_ = _gmm(be, na, xs, w_gate, W1, b1r, W2, b2r)
    y = _combine(os_, x2, pos0, pos1)
    return y.reshape(1, S, D), jnp.float32(0.0)


def kernel(x, mask, w_gate, W1, b1, W2, b2):
    return _moe(x, mask, w_gate, W1, b1, W2, b2)


# speedup vs baseline: 1.9655x; 1.0795x over previous
import functools

import jax
import jax.numpy as jnp
from jax import lax
from jax.experimental import pallas as pl
from jax.experimental.pallas import tpu as pltpu
from jax.experimental.pallas import tpu_sc as plsc

S, D, H, E = 2048, 1024, 1024, 16
TM = 256
NB = S * 2 // TM + E
PP = NB * TM
TT = 256
NTT = S // TT

NC, NS = 2, 16
NW = NC * NS
TPW = S // NW
SUB = 16

_mesh = plsc.VectorSubcoreMesh(core_axis_name="c", subcore_axis_name="s")


def _route_body(x_ref, wg_ref, pos0_ref, pos1_ref, be_ref, na_ref):
    lane = lambda shp: lax.broadcasted_iota(jnp.int32, shp, 1)
    tri = (lax.broadcasted_iota(jnp.int32, (TT, TT), 1)
           < lax.broadcasted_iota(jnp.int32, (TT, TT), 0)).astype(jnp.float32)

    def top2(ti):
        tok = pl.ds(ti * TT, TT)
        logits = jnp.dot(x_ref[tok, :], wg_ref[...],
                         preferred_element_type=jnp.float32)
        idx = lane(logits.shape)
        m0 = jnp.max(logits, axis=1, keepdims=True)
        i0 = jnp.min(jnp.where(logits == m0, idx, E), axis=1, keepdims=True)
        l2 = jnp.where(idx == i0, -1e30, logits)
        m1 = jnp.max(l2, axis=1, keepdims=True)
        i1 = jnp.min(jnp.where(l2 == m1, idx, E), axis=1, keepdims=True)
        return i0, i1

    carry = jnp.zeros((1, E), jnp.float32)
    rank0, oh0s, oh1s = [], [], []
    for ti in range(NTT):
        i0, i1 = top2(ti)
        oh0 = (lane((TT, E)) == i0).astype(jnp.float32)
        oh1 = (lane((TT, E)) == i1).astype(jnp.float32)
        cum = jnp.dot(tri, oh0, preferred_element_type=jnp.float32) + carry
        rank0.append(jnp.sum(jnp.where(lane((TT, E)) == i0, cum, 0.0),
                             axis=1, keepdims=True))
        carry = carry + jnp.sum(oh0, axis=0, keepdims=True)
        oh0s.append((i0, oh0))
        oh1s.append((i1, oh1))
    rank1 = []
    for ti in range(NTT):
        i1, oh1 = oh1s[ti]
        cum = jnp.dot(tri, oh1, preferred_element_type=jnp.float32) + carry
        rank1.append(jnp.sum(jnp.where(lane((TT, E)) == i1, cum, 0.0),
                             axis=1, keepdims=True))
        carry = carry + jnp.sum(oh1, axis=0, keepdims=True)
    cnt = carry
    nb = jnp.floor((cnt + (TM - 1)) * (1.0 / TM))
    tri_e = (lax.broadcasted_iota(jnp.int32, (E, E), 0)
             < lax.broadcasted_iota(jnp.int32, (E, E), 1)).astype(jnp.float32)
    bstart = jnp.dot(nb, tri_e, preferred_element_type=jnp.float32)
    pad_off = bstart * TM
    for ti in range(NTT):
        i0, _ = oh0s[ti]
        i1, _ = oh1s[ti]
        off0 = jnp.sum(jnp.where(lane((TT, E)) == i0, pad_off, 0.0),
                       axis=1, keepdims=True)
        off1 = jnp.sum(jnp.where(lane((TT, E)) == i1, pad_off, 0.0),
                       axis=1, keepdims=True)
        tok = pl.ds(ti * TT, TT)
        pos0_ref[tok, :] = (off0 + rank0[ti]).astype(jnp.int32)
        pos1_ref[tok, :] = (off1 + rank1[ti]).astype(jnp.int32)
    bidx = lax.broadcasted_iota(jnp.int32, (NB, E), 0).astype(jnp.float32)
    bst = jnp.broadcast_to(bstart, (NB, E))
    be_ref[...] = (jnp.sum((bst <= bidx).astype(jnp.float32), axis=1,
                           keepdims=True) - 1.0).astype(jnp.int32)
    na_ref[...] = jnp.sum(nb, axis=1, keepdims=True).astype(jnp.int32)


def _route(x2, w_gate):
    return pl.pallas_call(
        _route_body,
        grid=(),
        in_specs=[
            pl.BlockSpec((S, D), lambda: (0, 0)),
            pl.BlockSpec((D, E), lambda: (0, 0)),
        ],
        out_specs=[
            pl.BlockSpec((S, 1), lambda: (0, 0)),
            pl.BlockSpec((S, 1), lambda: (0, 0)),
            pl.BlockSpec((NB, 1), lambda: (0, 0)),
            pl.BlockSpec((1, 1), lambda: (0, 0)),
        ],
        out_shape=[
            jax.ShapeDtypeStruct((S, 1), jnp.int32),
            jax.ShapeDtypeStruct((S, 1), jnp.int32),
            jax.ShapeDtypeStruct((NB, 1), jnp.int32),
            jax.ShapeDtypeStruct((1, 1), jnp.int32),
        ],
    )(x2, w_gate)


DH = TPW // 2


@functools.partial(
    pl.kernel,
    out_type=jax.ShapeDtypeStruct((PP, D), jnp.float32),
    mesh=_mesh,
    scratch_types=[
        pltpu.VMEM((DH,), jnp.int32),
        pltpu.VMEM((DH,), jnp.int32),
        pltpu.VMEM((DH,), jnp.int32),
        pltpu.VMEM((DH,), jnp.int32),
        pltpu.VMEM((DH, D), jnp.float32),
        pltpu.VMEM((DH, D), jnp.float32),
        pltpu.SemaphoreType.DMA,
        pltpu.SemaphoreType.DMA,
        pltpu.SemaphoreType.DMA,
        pltpu.SemaphoreType.DMA,
        pltpu.SemaphoreType.DMA,
        pltpu.SemaphoreType.DMA,
    ],
)
def _dispatch(x_hbm, pos0_hbm, pos1_hbm, xs_hbm, i0a_v, i1a_v, i0b_v, i1b_v,
              xa_v, xb_v, sla, slb, s0a, s1a, s0b, s1b):
    wid = lax.axis_index("s") * NC + lax.axis_index("c")
    base = wid * TPW
    la = pltpu.async_copy(x_hbm.at[pl.ds(base, DH)], xa_v, sla)
    lb = pltpu.async_copy(x_hbm.at[pl.ds(base + DH, DH)], xb_v, slb)
    pltpu.sync_copy(pos0_hbm.at[pl.ds(base, DH)], i0a_v)
    pltpu.sync_copy(pos1_hbm.at[pl.ds(base, DH)], i1a_v)
    pltpu.sync_copy(pos0_hbm.at[pl.ds(base + DH, DH)], i0b_v)
    pltpu.sync_copy(pos1_hbm.at[pl.ds(base + DH, DH)], i1b_v)
    la.wait()
    c0a = pltpu.async_copy(xa_v, xs_hbm.at[i0a_v], s0a)
    c1a = pltpu.async_copy(xa_v, xs_hbm.at[i1a_v], s1a)
    lb.wait()
    c0b = pltpu.async_copy(xb_v, xs_hbm.at[i0b_v], s0b)
    c1b = pltpu.async_copy(xb_v, xs_hbm.at[i1b_v], s1b)
    c0a.wait()
    c1a.wait()
    c0b.wait()
    c1b.wait()


def _gmm_body(be_ref, na_ref, xs_ref, wg_ref, W1_ref, b1_ref, W2_ref, b2_ref,
              out_ref):
    b = pl.program_id(0)

    @pl.when(b < na_ref[0])
    def _():
        xf = xs_ref[...]
        e = be_ref[b]
        logits = jnp.dot(xf, wg_ref[...], preferred_element_type=jnp.float32)
        idx = lax.broadcasted_iota(jnp.int32, logits.shape, 1)
        m0 = jnp.max(logits, axis=1, keepdims=True)
        i0 = jnp.min(jnp.where(logits == m0, idx, E), axis=1, keepdims=True)
        l2 = jnp.where(idx == i0, -1e30, logits)
        m1 = jnp.max(l2, axis=1, keepdims=True)
        i1 = jnp.min(jnp.where(l2 == m1, idx, E), axis=1, keepdims=True)
        g0 = 1.0 / (1.0 + jnp.exp(m1 - m0))
        g = jnp.where(i0 == e, g0, 0.0) + jnp.where(i1 == e, 1.0 - g0, 0.0)

        xb = xf.astype(jnp.bfloat16)
        h = jnp.dot(xb, W1_ref[0].astype(jnp.bfloat16),
                    preferred_element_type=jnp.float32)
        h = jnp.maximum(h + b1_ref[0], 0.0).astype(jnp.bfloat16)
        o = jnp.dot(h, W2_ref[0].astype(jnp.bfloat16),
                    preferred_element_type=jnp.float32)
        out_ref[...] = g * (o + b2_ref[0])


def _gmm(be, na, xs, w_gate, W1b, b1r, W2b, b2r):
    def _b(b, be, na):
        return jnp.minimum(b, na[0] - 1)

    grid_spec = pltpu.PrefetchScalarGridSpec(
        num_scalar_prefetch=2,
        grid=(NB,),
        in_specs=[
            pl.BlockSpec((TM, D), lambda b, be, na: (_b(b, be, na), 0)),
            pl.BlockSpec((D, E), lambda b, be, na: (0, 0)),
            pl.BlockSpec((1, D, H), lambda b, be, na: (be[_b(b, be, na)], 0, 0)),
            pl.BlockSpec((1, 1, H), lambda b, be, na: (be[_b(b, be, na)], 0, 0)),
            pl.BlockSpec((1, H, D), lambda b, be, na: (be[_b(b, be, na)], 0, 0)),
            pl.BlockSpec((1, 1, D), lambda b, be, na: (be[_b(b, be, na)], 0, 0)),
        ],
        out_specs=pl.BlockSpec(
            (TM, D), lambda b, be, na: (_b(b, be, na), 0)),
    )
    return pl.pallas_call(
        _gmm_body,
        grid_spec=grid_spec,
        out_shape=jax.ShapeDtypeStruct((PP, D), jnp.float32),
        compiler_params=pltpu.CompilerParams(
            dimension_semantics=("arbitrary",)),
    )(be, na, xs, w_gate, W1b, b1r, W2b, b2r)


NCH = TPW // SUB


@functools.partial(
    pl.kernel,
    out_type=jax.ShapeDtypeStruct((S, D), jnp.float32),
    mesh=_mesh,
    scratch_types=[
        pltpu.VMEM((TPW,), jnp.int32),
        pltpu.VMEM((TPW,), jnp.int32),
        pltpu.VMEM((SUB, D), jnp.float32),
        pltpu.VMEM((SUB, D), jnp.float32),
        pltpu.VMEM((SUB, D), jnp.float32),
        pltpu.VMEM((SUB, D), jnp.float32),
        pltpu.VMEM((SUB, D), jnp.float32),
        pltpu.VMEM((SUB, D), jnp.float32),
        pltpu.SemaphoreType.DMA,
        pltpu.SemaphoreType.DMA,
        pltpu.SemaphoreType.DMA,
        pltpu.SemaphoreType.DMA,
        pltpu.SemaphoreType.DMA,
        pltpu.SemaphoreType.DMA,
        pltpu.SemaphoreType.DMA,
        pltpu.SemaphoreType.DMA,
    ],
)
def _combine(os_hbm, x_hbm, pos0_hbm, pos1_hbm, y_hbm,
             idx0_v, idx1_v, ab0, ab1, bb0, bb1, xb0, xb1,
             sa0, sa1, sb0, sb1, sx0, sx1, sy0, sy1):
    wid = lax.axis_index("s") * NC + lax.axis_index("c")
    base = wid * TPW
    pltpu.sync_copy(pos0_hbm.at[pl.ds(base, TPW)], idx0_v)
    pltpu.sync_copy(pos1_hbm.at[pl.ds(base, TPW)], idx1_v)
    bufs = [(ab0, bb0, xb0, sa0, sb0, sx0, sy0),
            (ab1, bb1, xb1, sa1, sb1, sx1, sy1)]

    def start_loads(j):
        ab, bb, xb, sa, sb, sx, _ = bufs[j & 1]
        i0 = idx0_v[pl.ds(j * SUB, SUB)]
        i1 = idx1_v[pl.ds(j * SUB, SUB)]
        ca = pltpu.async_copy(os_hbm.at[i0], ab, sa)
        cb = pltpu.async_copy(os_hbm.at[i1], bb, sb)
        cx = pltpu.async_copy(x_hbm.at[pl.ds(base + j * SUB, SUB)], xb, sx)
        return ca, cb, cx

    pend = [None, None]
    ystore = [None, None]
    pend[0] = start_loads(0)
    for j in range(NCH):
        cur = j & 1
        nxt = 1 - cur
        if j + 1 < NCH:
            if ystore[nxt] is not None:
                ystore[nxt].wait()
                ystore[nxt] = None
            pend[nxt] = start_loads(j + 1)
        ca, cb, cx = pend[cur]
        ca.wait()
        cb.wait()
        cx.wait()
        ab, bb, xb, _, _, _, sy = bufs[cur]

        def col(c, _):
            sl = pl.ds(c * 16, 16)
            for t in range(SUB):
                xb[t, sl] = xb[t, sl] + ab[t, sl] + bb[t, sl]
            return _

        lax.fori_loop(0, D // 16, col, 0)
        ystore[cur] = pltpu.async_copy(
            xb, y_hbm.at[pl.ds(base + j * SUB, SUB)], sy)
    for st in ystore:
        if st is not None:
            st.wait()


@jax.jit
def _moe(x, mask, w_gate, W1, b1, W2, b2):
    x2 = x.reshape(S, D)
    b1r = b1.reshape(E, 1, H)
    b2r = b2.reshape(E, 1, D)

    pos0, pos1, be, na = _route(x2, w_gate)
    pos0 = pos0.reshape(S)
    pos1 = pos1.reshape(S)
    be = be.reshape(NB)
    na = na.reshape(1)

    xs = _dispatch(x2, pos0, pos1)
    os_ = _gmm(be, na, xs, w_gate, W1, b1r, W2, b2r)
    y = _combine(os_, x2, pos0, pos1)
    return y.reshape(1, S, D), jnp.float32(0.0)


def kernel(x, mask, w_gate, W1, b1, W2, b2):
    return _moe(x, mask, w_gate, W1, b1, W2, b2)
